# Initial kernel scaffold; baseline (speedup 1.0000x reference)
import jax
import jax.numpy as jnp
from jax.experimental import pallas as pl


def kernel(features):
    raise NotImplementedError("write your pallas kernel here")


# speedup vs baseline: 3.1886x; 3.1886x over previous
import jax
import jax.numpy as jnp
from jax.experimental import pallas as pl

_NUM_SLOTS = 9


def _fill_kernel(in_ref, out_ref):
    j = pl.program_id(1)
    is_copy = jnp.logical_or(j == 0, j == _NUM_SLOTS - 1)

    @pl.when(is_copy)
    def _():
        out_ref[...] = in_ref[...]

    @pl.when(jnp.logical_not(is_copy))
    def _():
        out_ref[...] = jnp.zeros_like(out_ref)


def kernel(features):
    B = features.shape[0]
    feat2d = features.reshape(B, -1)
    F = feat2d.shape[1]
    rb = 16
    return pl.pallas_call(
        _fill_kernel,
        grid=(B // rb, _NUM_SLOTS),
        in_specs=[pl.BlockSpec((rb, F), lambda i, j: (i, 0))],
        out_specs=pl.BlockSpec((rb, F), lambda i, j: (i, j)),
        out_shape=jax.ShapeDtypeStruct((B, _NUM_SLOTS * F), feat2d.dtype),
    )(feat2d)


# speedup vs baseline: 3.6678x; 1.1503x over previous
import jax
import jax.numpy as jnp
from jax.experimental import pallas as pl

_NUM_SLOTS = 9


def _fill_kernel(in_ref, out_ref):
    j = pl.program_id(1)
    is_copy = jnp.logical_or(j == 0, j == _NUM_SLOTS - 1)

    @pl.when(is_copy)
    def _():
        out_ref[...] = in_ref[...]

    @pl.when(jnp.logical_not(is_copy))
    def _():
        out_ref[...] = jnp.zeros_like(out_ref)


def kernel(features):
    B = features.shape[0]
    feat2d = features.reshape(B, -1)
    F = feat2d.shape[1]
    rb = 32
    return pl.pallas_call(
        _fill_kernel,
        grid=(B // rb, _NUM_SLOTS),
        in_specs=[pl.BlockSpec((rb, F), lambda i, j: (i, 0))],
        out_specs=pl.BlockSpec((rb, F), lambda i, j: (i, j)),
        out_shape=jax.ShapeDtypeStruct((B, _NUM_SLOTS * F), feat2d.dtype),
    )(feat2d)


# speedup vs baseline: 3.7577x; 1.0245x over previous
import jax
import jax.numpy as jnp
from jax.experimental import pallas as pl

_NUM_SLOTS = 9


def _fill_kernel(in_ref, out_ref):
    j = pl.program_id(1)
    is_copy = jnp.logical_or(j == 0, j == _NUM_SLOTS - 1)

    @pl.when(is_copy)
    def _():
        out_ref[...] = in_ref[...]

    @pl.when(jnp.logical_not(is_copy))
    def _():
        out_ref[...] = jnp.zeros_like(out_ref)


def kernel(features):
    B = features.shape[0]
    feat2d = features.reshape(B, -1)
    F = feat2d.shape[1]
    rb = 64
    return pl.pallas_call(
        _fill_kernel,
        grid=(B // rb, _NUM_SLOTS),
        in_specs=[pl.BlockSpec((rb, F), lambda i, j: (i, 0))],
        out_specs=pl.BlockSpec((rb, F), lambda i, j: (i, j)),
        out_shape=jax.ShapeDtypeStruct((B, _NUM_SLOTS * F), feat2d.dtype),
    )(feat2d)


# speedup vs baseline: 3.9306x; 1.0460x over previous
import jax
import jax.numpy as jnp
from jax.experimental import pallas as pl

_NUM_SLOTS = 9


def _fill_kernel(in_ref, out_ref):
    F = in_ref.shape[1]
    out_ref[:, :F] = in_ref[...]
    out_ref[:, F:(_NUM_SLOTS - 1) * F] = jnp.zeros_like(
        out_ref[:, F:(_NUM_SLOTS - 1) * F]
    )
    out_ref[:, (_NUM_SLOTS - 1) * F:] = in_ref[...]


def kernel(features):
    B = features.shape[0]
    feat2d = features.reshape(B, -1)
    F = feat2d.shape[1]
    rb = 16
    return pl.pallas_call(
        _fill_kernel,
        grid=(B // rb,),
        in_specs=[pl.BlockSpec((rb, F), lambda i: (i, 0))],
        out_specs=pl.BlockSpec((rb, _NUM_SLOTS * F), lambda i: (i, 0)),
        out_shape=jax.ShapeDtypeStruct((B, _NUM_SLOTS * F), feat2d.dtype),
    )(feat2d)
